# baseline (speedup 1.0000x reference)
import functools

import jax
import jax.numpy as jnp
import numpy as np
from jax import lax
from jax.experimental import pallas as pl
from jax.experimental.pallas import tpu as pltpu
from jax.experimental.pallas import tpu_sc as plsc

_N = 100000
_E = 1600000
_G = 512
_H = 32

_EPAD = 1605632
_NROW = _EPAD // 128

_PT1_ROWS = _NROW // 32
_NB1 = 8
_NC1 = _PT1_ROWS // _NB1

_PT2_ROWS = _NROW // 16
_NB2 = 4
_NC2 = _PT2_ROWS // _NB2

_NDPAD = 100096
_NDPT = _NDPAD // 16

_mesh = plsc.VectorSubcoreMesh(core_axis_name="c", subcore_axis_name="s")
_sc_params = pltpu.CompilerParams(use_tc_tiling_on_sc=False)


def _pg_body(hrows_h, src_h, dst_h, et_h, s_h, t_h,
             accA_h, accB_h, denA_h, denB_h,
             srcv, dstv, gidxv, sv, tv, etv, exv, rowsbuf,
             shacc, shden, semg, sems):
    c = lax.axis_index("c")
    sid = lax.axis_index("s")
    lanes = lax.iota(jnp.int32, 16)

    def zfill(i, _):
        rowsbuf[i] = jnp.zeros((16,), jnp.float32)
        return 0
    lax.fori_loop(0, 368, zfill, 0)

    def zfilld(i, _):
        exv[pl.ds(i * 16, 16)] = jnp.zeros((16,), jnp.float32)
        return 0
    lax.fori_loop(0, 23, zfilld, 0)

    zb = rowsbuf.at[pl.ds(0, 368)]
    zbd = exv.at[pl.ds(0, 368)]

    def zcopy(j, _):
        pltpu.sync_copy(zb, shacc.at[pl.ds(sid * _NDPT + j * 368, 368)])
        pltpu.sync_copy(zbd, shden.at[pl.ds(sid * _NDPT + j * 368, 368)])
        return 0
    lax.fori_loop(0, 17, zcopy, 0)
    plsc.subcore_barrier()

    def stage_fire(nc, hh):
        r0 = sid * _PT2_ROWS + nc * _NB2
        pltpu.sync_copy(src_h.at[pl.ds(r0, _NB2)], srcv)
        pltpu.sync_copy(dst_h.at[pl.ds(r0, _NB2)],
                        dstv.at[pl.ds(hh * _NB2, _NB2)])
        pltpu.async_copy(et_h.at[pl.ds(r0 * 128, _NB2 * 128)], etv, semg)
        for b in range(_NB2):
            pltpu.async_copy(s_h.at[srcv.at[b]], sv.at[b], semg)
            pltpu.async_copy(t_h.at[dstv.at[hh * _NB2 + b]], tv.at[b], semg)
        for b in range(_NB2):
            for g in range(8):
                sl = pl.ds(g * 16, 16)
                gidxv[b, sl] = srcv[b, sl] * 2 + c
        for b in range(_NB2):
            pltpu.async_copy(
                hrows_h.at[gidxv.at[b]],
                rowsbuf.at[pl.ds(hh * (_NB2 * 128) + b * 128, 128)], semg)

    def wait_gathers():
        pltpu.make_async_copy(
            et_h.at[pl.ds(0, _NB2 * 128)], etv, semg).wait()
        for b in range(_NB2):
            pltpu.make_async_copy(s_h.at[srcv.at[b]], sv.at[b], semg).wait()
            pltpu.make_async_copy(s_h.at[srcv.at[b]], tv.at[b], semg).wait()
        for b in range(_NB2):
            pltpu.make_async_copy(
                hrows_h.at[gidxv.at[b]],
                rowsbuf.at[pl.ds(b * 128, 128)], semg).wait()

    def drain_scatters(hh):
        for b in range(_NB2):
            pltpu.make_async_copy(
                rowsbuf.at[pl.ds(hh * (_NB2 * 128) + b * 128, 128)],
                shacc.at[dstv.at[hh * _NB2 + b]], sems).wait()

        @pl.when(c == hh)
        def _():
            for b in range(_NB2):
                pltpu.make_async_copy(
                    exv.at[pl.ds(hh * (_NB2 * 128) + b * 128, 128)],
                    shden.at[dstv.at[hh * _NB2 + b]], sems).wait()

    stage_fire(0, 0)

    def chunk(ci, _):
        hh = lax.rem(ci, 2)
        ho = hh * (_NB2 * 128)
        r0 = sid * _PT2_ROWS + ci * _NB2
        wait_gathers()
        for b in range(_NB2):
            for g in range(8):
                sl16 = pl.ds(g * 16, 16)
                a = sv[b, sl16] + tv[b, sl16] + etv[pl.ds(b * 128 + g * 16, 16)]
                a = jnp.where(a > 0, a, a * np.float32(0.2))
                ge = (r0 + b) * 128 + g * 16 + lanes
                exv[pl.ds(ho + b * 128 + g * 16, 16)] = jnp.where(
                    ge < _E, jnp.exp(a), np.float32(0.0))

        def scale(q, _):
            exq = exv[pl.ds(ho + q * 16, 16)]
            for l in range(16):
                row = ho + q * 16 + l
                rowsbuf[row] = rowsbuf[row] * exq[l]
            return 0
        lax.fori_loop(0, _NB2 * 8, scale, 0)

        @pl.when(ci > 0)
        def _():
            drain_scatters(1 - hh)

        for b in range(_NB2):
            pltpu.async_copy(rowsbuf.at[pl.ds(ho + b * 128, 128)],
                             shacc.at[dstv.at[hh * _NB2 + b]], sems, add=True)

        @pl.when(c == hh)
        def _():
            for b in range(_NB2):
                pltpu.async_copy(exv.at[pl.ds(ho + b * 128, 128)],
                                 shden.at[dstv.at[hh * _NB2 + b]], sems,
                                 add=True)

        @pl.when(ci + 1 < _NC2)
        def _():
            stage_fire(ci + 1, 1 - hh)
        return 0

    lax.fori_loop(0, _NC2, chunk, 0)
    drain_scatters(lax.rem(_NC2 - 1, 2))
    plsc.subcore_barrier()

    @pl.when(c == 0)
    def _():
        pltpu.sync_copy(shacc.at[pl.ds(sid * _NDPT, _NDPT)],
                        accA_h.at[pl.ds(sid * _NDPT, _NDPT)])
        pltpu.sync_copy(shden.at[pl.ds(sid * _NDPT, _NDPT)],
                        denA_h.at[pl.ds(sid * _NDPT, _NDPT)])

    @pl.when(c == 1)
    def _():
        pltpu.sync_copy(shacc.at[pl.ds(sid * _NDPT, _NDPT)],
                        accB_h.at[pl.ds(sid * _NDPT, _NDPT)])
        pltpu.sync_copy(shden.at[pl.ds(sid * _NDPT, _NDPT)],
                        denB_h.at[pl.ds(sid * _NDPT, _NDPT)])


_pgat = functools.partial(
    pl.kernel,
    _pg_body,
    out_type=(
        jax.ShapeDtypeStruct((_NDPAD, 16), jnp.float32),
        jax.ShapeDtypeStruct((_NDPAD, 16), jnp.float32),
        jax.ShapeDtypeStruct((_NDPAD,), jnp.float32),
        jax.ShapeDtypeStruct((_NDPAD,), jnp.float32),
    ),
    mesh=_mesh,
    scratch_types=[
        pltpu.VMEM((_NB2, 128), jnp.int32),
        pltpu.VMEM((2 * _NB2, 128), jnp.int32),
        pltpu.VMEM((_NB2, 128), jnp.int32),
        pltpu.VMEM((_NB2, 128), jnp.float32),
        pltpu.VMEM((_NB2, 128), jnp.float32),
        pltpu.VMEM((_NB2 * 128,), jnp.float32),
        pltpu.VMEM((2 * _NB2 * 128,), jnp.float32),
        pltpu.VMEM((2 * _NB2 * 128, 16), jnp.float32),
        pltpu.VMEM_SHARED((_NDPAD, 16), jnp.float32),
        pltpu.VMEM_SHARED((_NDPAD,), jnp.float32),
        pltpu.SemaphoreType.DMA,
        pltpu.SemaphoreType.DMA,
    ],
    compiler_params=_sc_params,
)()


_BLK = 5000


def _t0_body(x_ref, w_ref, as_ref, ad_ref, h_ref, s_ref, t_ref):
    h = jnp.dot(x_ref[...], w_ref[...], preferred_element_type=jnp.float32)
    h_ref[...] = h
    s_ref[...] = jnp.dot(h, as_ref[...], preferred_element_type=jnp.float32)
    t_ref[...] = jnp.dot(h, ad_ref[...], preferred_element_type=jnp.float32)


def _t0(xp, Wp, a_s, a_d):
    return pl.pallas_call(
        _t0_body,
        grid=(_N // _BLK,),
        in_specs=[
            pl.BlockSpec((_BLK, 8), lambda i: (i, 0)),
            pl.BlockSpec((8, _H), lambda i: (0, 0)),
            pl.BlockSpec((_H, 1), lambda i: (0, 0)),
            pl.BlockSpec((_H, 1), lambda i: (0, 0)),
        ],
        out_specs=[
            pl.BlockSpec((_BLK, _H), lambda i: (i, 0)),
            pl.BlockSpec((_BLK, 1), lambda i: (i, 0)),
            pl.BlockSpec((_BLK, 1), lambda i: (i, 0)),
        ],
        out_shape=[
            jax.ShapeDtypeStruct((_N, _H), jnp.float32),
            jax.ShapeDtypeStruct((_N, 1), jnp.float32),
            jax.ShapeDtypeStruct((_N, 1), jnp.float32),
        ],
    )(xp, Wp, a_s, a_d)


_BLKE = _EPAD // 16


def _te_body(w4p_ref, eat_ref, out_ref):
    out_ref[...] = jnp.dot(w4p_ref[...], eat_ref[...],
                           preferred_element_type=jnp.float32)


def _te(w4p, eaT8):
    return pl.pallas_call(
        _te_body,
        grid=(_EPAD // _BLKE,),
        in_specs=[
            pl.BlockSpec((8, 8), lambda i: (0, 0)),
            pl.BlockSpec((8, _BLKE), lambda i: (0, i)),
        ],
        out_specs=pl.BlockSpec((8, _BLKE), lambda i: (0, i)),
        out_shape=jax.ShapeDtypeStruct((8, _EPAD), jnp.float32),
    )(w4p, eaT8)


def _t1_body(accA_ref, accB_ref, dena_ref, denb_ref, b_ref, w_ref, as_ref,
             ad_ref, h_ref, s_ref, t_ref, *, do_relu):
    acc = jnp.concatenate([accA_ref[...], accB_ref[...]], axis=1)
    den = jnp.maximum(dena_ref[...] + denb_ref[...], np.float32(1e-16))
    o = acc / den + b_ref[...]
    if do_relu:
        o = jnp.maximum(o, np.float32(0.0))
    h = jnp.dot(o, w_ref[...], preferred_element_type=jnp.float32)
    h_ref[...] = h
    s_ref[...] = jnp.dot(h, as_ref[...], preferred_element_type=jnp.float32)
    t_ref[...] = jnp.dot(h, ad_ref[...], preferred_element_type=jnp.float32)


def _t1(accA, accB, dena, denb, b, W, a_s, a_d, do_relu):
    return pl.pallas_call(
        functools.partial(_t1_body, do_relu=do_relu),
        grid=(_N // _BLK,),
        in_specs=[
            pl.BlockSpec((_BLK, 16), lambda i: (i, 0)),
            pl.BlockSpec((_BLK, 16), lambda i: (i, 0)),
            pl.BlockSpec((_BLK, 1), lambda i: (i, 0)),
            pl.BlockSpec((_BLK, 1), lambda i: (i, 0)),
            pl.BlockSpec((1, _H), lambda i: (0, 0)),
            pl.BlockSpec((_H, _H), lambda i: (0, 0)),
            pl.BlockSpec((_H, 1), lambda i: (0, 0)),
            pl.BlockSpec((_H, 1), lambda i: (0, 0)),
        ],
        out_specs=[
            pl.BlockSpec((_BLK, _H), lambda i: (i, 0)),
            pl.BlockSpec((_BLK, 1), lambda i: (i, 0)),
            pl.BlockSpec((_BLK, 1), lambda i: (i, 0)),
        ],
        out_shape=[
            jax.ShapeDtypeStruct((_N, _H), jnp.float32),
            jax.ShapeDtypeStruct((_N, 1), jnp.float32),
            jax.ShapeDtypeStruct((_N, 1), jnp.float32),
        ],
    )(accA, accB, dena, denb, b, W, a_s, a_d)


_BLKH = 2000
_NBH = _N // _BLKH


def _head_body(accA_ref, accB_ref, dena_ref, denb_ref, b_ref, batch_ref,
               wl_ref, bl_ref, out_ref, pooled, cnt):
    i = pl.program_id(0)

    @pl.when(i == 0)
    def _():
        pooled[...] = jnp.zeros_like(pooled)
        cnt[...] = jnp.zeros_like(cnt)

    acc = jnp.concatenate([accA_ref[...], accB_ref[...]], axis=1)
    den = jnp.maximum(dena_ref[...] + denb_ref[...], np.float32(1e-16))
    h3 = acc / den + b_ref[...]
    bb = batch_ref[0]
    gids = lax.broadcasted_iota(jnp.int32, (_G, 1), 0)
    onehotT = (gids == bb).astype(jnp.float32)
    pooled[...] += jnp.dot(onehotT, h3, preferred_element_type=jnp.float32)
    cnt[...] += jnp.dot(onehotT, jnp.ones((_BLKH, 8), jnp.float32),
                        preferred_element_type=jnp.float32)

    @pl.when(i == _NBH - 1)
    def _():
        gm = pooled[...] / jnp.maximum(cnt[..., 0:1], np.float32(1.0))
        logits = jnp.dot(gm, wl_ref[...], preferred_element_type=jnp.float32)
        logits = logits + bl_ref[...]
        mx = jnp.max(logits, axis=1, keepdims=True)
        e = jnp.exp(logits - mx)
        out_ref[...] = e / jnp.sum(e, axis=1, keepdims=True)


def _head(accA, accB, dena, denb, b, batch3, Wlp, blp):
    return pl.pallas_call(
        _head_body,
        grid=(_NBH,),
        in_specs=[
            pl.BlockSpec((_BLKH, 16), lambda i: (i, 0)),
            pl.BlockSpec((_BLKH, 16), lambda i: (i, 0)),
            pl.BlockSpec((_BLKH, 1), lambda i: (i, 0)),
            pl.BlockSpec((_BLKH, 1), lambda i: (i, 0)),
            pl.BlockSpec((1, _H), lambda i: (0, 0)),
            pl.BlockSpec((1, 1, _BLKH), lambda i: (i, 0, 0)),
            pl.BlockSpec((_H, 8), lambda i: (0, 0)),
            pl.BlockSpec((1, 8), lambda i: (0, 0)),
        ],
        out_specs=pl.BlockSpec((_G, 8), lambda i: (0, 0)),
        out_shape=jax.ShapeDtypeStruct((_G, 8), jnp.float32),
        scratch_shapes=[
            pltpu.VMEM((_G, _H), jnp.float32),
            pltpu.VMEM((_G, 8), jnp.float32),
        ],
    )(accA, accB, dena, denb, b, batch3, Wlp, blp)


def kernel(x, edge_index, edge_attr, batch, W1, a_s1, a_d1, We1, a_e1, b1,
           W2, a_s2, a_d2, We2, a_e2, b2, W3, a_s3, a_d3, We3, a_e3, b3,
           Wl, bl):
    f32 = jnp.float32
    pad_e = _EPAD - _E
    src = jnp.pad(edge_index[0], (0, pad_e)).reshape(_NROW, 128)
    dst = jnp.pad(edge_index[1], (0, pad_e)).reshape(_NROW, 128)
    eaT8 = jnp.pad(jnp.pad(edge_attr, ((0, pad_e), (0, 0))).T,
                   ((0, 4), (0, 0)))
    xp = jnp.pad(x, ((0, 0), (0, 1)))
    W1p = jnp.pad(W1, ((0, 1), (0, 0)))
    batch3 = batch.reshape(_NBH, 1, _BLKH)
    Wlp = jnp.pad(Wl, ((0, 0), (0, 6)))
    blp = jnp.concatenate([bl, jnp.full((6,), -1e30, f32)]).reshape(1, 8)

    w4p = jnp.zeros((8, 8), f32)
    w4p = w4p.at[0, :4].set(We1 @ a_e1)
    w4p = w4p.at[1, :4].set(We2 @ a_e2)
    w4p = w4p.at[2, :4].set(We3 @ a_e3)
    et8 = _te(w4p, eaT8)

    layers = [
        (W1p, a_s1, a_d1, b1),
        (W2, a_s2, a_d2, b2),
        (W3, a_s3, a_d3, b3),
    ]

    h, s, t = _t0(xp, W1p, a_s1.reshape(_H, 1), a_d1.reshape(_H, 1))
    for li in range(3):
        _, _, _, b = layers[li]
        accA, accB, denA, denB = _pgat(h.reshape(2 * _N, 16), src, dst,
                                       et8[li], s.reshape(_N), t.reshape(_N))
        accA, accB = accA[:_N], accB[:_N]
        denA2 = denA[:_N].reshape(_N, 1)
        denB2 = denB[:_N].reshape(_N, 1)
        if li < 2:
            Wn, a_sn, a_dn, _ = layers[li + 1]
            h, s, t = _t1(accA, accB, denA2, denB2, b.reshape(1, _H), Wn,
                          a_sn.reshape(_H, 1), a_dn.reshape(_H, 1),
                          do_relu=True)
        else:
            out = _head(accA, accB, denA2, denB2, b.reshape(1, _H), batch3,
                        Wlp, blp)
    return out[:, :2]


# speedup vs baseline: 1.0012x; 1.0012x over previous
import functools

import jax
import jax.numpy as jnp
import numpy as np
from jax import lax
from jax.experimental import pallas as pl
from jax.experimental.pallas import tpu as pltpu
from jax.experimental.pallas import tpu_sc as plsc

_N = 100000
_E = 1600000
_G = 512
_H = 32

_EPAD = 1605632
_NROW = _EPAD // 128

_PT1_ROWS = _NROW // 32
_NB1 = 8
_NC1 = _PT1_ROWS // _NB1

_PT2_ROWS = _NROW // 16
_NB2 = 4
_NC2 = _PT2_ROWS // _NB2

_NDPAD = 100096
_NDPT = _NDPAD // 16

_mesh = plsc.VectorSubcoreMesh(core_axis_name="c", subcore_axis_name="s")
_sc_params = pltpu.CompilerParams(use_tc_tiling_on_sc=False)


def _pg_body(hrows_h, src_h, dst_h, et_h, s_h, t_h,
             accA_h, accB_h, denA_h, denB_h,
             srcv, dstv, gidxv, sv, tv, etv, exv, rowsbuf,
             shacc, shden, semg, sems):
    c = lax.axis_index("c")
    sid = lax.axis_index("s")
    lanes = lax.iota(jnp.int32, 16)

    def zfill(i, _):
        rowsbuf[i] = jnp.zeros((16,), jnp.float32)
        return 0
    lax.fori_loop(0, 368, zfill, 0)

    def zfilld(i, _):
        exv[pl.ds(i * 16, 16)] = jnp.zeros((16,), jnp.float32)
        return 0
    lax.fori_loop(0, 23, zfilld, 0)

    zb = rowsbuf.at[pl.ds(0, 368)]
    zbd = exv.at[pl.ds(0, 368)]

    def zcopy(j, _):
        pltpu.sync_copy(zb, shacc.at[pl.ds(sid * _NDPT + j * 368, 368)])
        pltpu.sync_copy(zbd, shden.at[pl.ds(sid * _NDPT + j * 368, 368)])
        return 0
    lax.fori_loop(0, 17, zcopy, 0)
    plsc.subcore_barrier()

    def stage_fire(nc, hh):
        r0 = sid * _PT2_ROWS + nc * _NB2
        pltpu.sync_copy(src_h.at[pl.ds(r0, _NB2)], srcv)
        pltpu.sync_copy(dst_h.at[pl.ds(r0, _NB2)],
                        dstv.at[pl.ds(hh * _NB2, _NB2)])
        pltpu.async_copy(et_h.at[pl.ds(r0 * 128, _NB2 * 128)], etv, semg)
        for b in range(_NB2):
            pltpu.async_copy(s_h.at[srcv.at[b]], sv.at[b], semg)
            pltpu.async_copy(t_h.at[dstv.at[hh * _NB2 + b]], tv.at[b], semg)
        for b in range(_NB2):
            for g in range(8):
                sl = pl.ds(g * 16, 16)
                gidxv[b, sl] = srcv[b, sl] * 2 + c
        for b in range(_NB2):
            pltpu.async_copy(
                hrows_h.at[gidxv.at[b]],
                rowsbuf.at[pl.ds(hh * (_NB2 * 128) + b * 128, 128)], semg)

    def wait_gathers():
        pltpu.make_async_copy(
            et_h.at[pl.ds(0, _NB2 * 128)], etv, semg).wait()
        for b in range(_NB2):
            pltpu.make_async_copy(s_h.at[srcv.at[b]], sv.at[b], semg).wait()
            pltpu.make_async_copy(s_h.at[srcv.at[b]], tv.at[b], semg).wait()
        for b in range(_NB2):
            pltpu.make_async_copy(
                hrows_h.at[gidxv.at[b]],
                rowsbuf.at[pl.ds(b * 128, 128)], semg).wait()

    def drain_scatters(hh):
        for b in range(_NB2):
            pltpu.make_async_copy(
                rowsbuf.at[pl.ds(hh * (_NB2 * 128) + b * 128, 128)],
                shacc.at[dstv.at[hh * _NB2 + b]], sems).wait()

        @pl.when(c == 0)
        def _():
            for b in range(_NB2):
                pltpu.make_async_copy(
                    exv.at[pl.ds(hh * (_NB2 * 128) + b * 128, 128)],
                    shden.at[dstv.at[hh * _NB2 + b]], sems).wait()

    stage_fire(0, 0)

    def chunk(ci, _):
        hh = lax.rem(ci, 2)
        ho = hh * (_NB2 * 128)
        r0 = sid * _PT2_ROWS + ci * _NB2
        wait_gathers()
        for b in range(_NB2):
            for g in range(8):
                sl16 = pl.ds(g * 16, 16)
                a = sv[b, sl16] + tv[b, sl16] + etv[pl.ds(b * 128 + g * 16, 16)]
                a = jnp.where(a > 0, a, a * np.float32(0.2))
                ge = (r0 + b) * 128 + g * 16 + lanes
                exv[pl.ds(ho + b * 128 + g * 16, 16)] = jnp.where(
                    ge < _E, jnp.exp(a), np.float32(0.0))

        def scale(q, _):
            exq = exv[pl.ds(ho + q * 16, 16)]
            for l in range(16):
                row = ho + q * 16 + l
                rowsbuf[row] = rowsbuf[row] * exq[l]
            return 0
        lax.fori_loop(0, _NB2 * 8, scale, 0)

        @pl.when(ci > 0)
        def _():
            drain_scatters(1 - hh)

        for b in range(_NB2):
            pltpu.async_copy(rowsbuf.at[pl.ds(ho + b * 128, 128)],
                             shacc.at[dstv.at[hh * _NB2 + b]], sems, add=True)

        @pl.when(c == 0)
        def _():
            for b in range(_NB2):
                pltpu.async_copy(exv.at[pl.ds(ho + b * 128, 128)],
                                 shden.at[dstv.at[hh * _NB2 + b]], sems,
                                 add=True)

        @pl.when(ci + 1 < _NC2)
        def _():
            stage_fire(ci + 1, 1 - hh)
        return 0

    lax.fori_loop(0, _NC2, chunk, 0)
    drain_scatters(lax.rem(_NC2 - 1, 2))
    plsc.subcore_barrier()

    @pl.when(c == 0)
    def _():
        pltpu.sync_copy(shacc.at[pl.ds(sid * _NDPT, _NDPT)],
                        accA_h.at[pl.ds(sid * _NDPT, _NDPT)])
        pltpu.sync_copy(shden.at[pl.ds(sid * _NDPT, _NDPT)],
                        denA_h.at[pl.ds(sid * _NDPT, _NDPT)])

    @pl.when(c == 1)
    def _():
        pltpu.sync_copy(shacc.at[pl.ds(sid * _NDPT, _NDPT)],
                        accB_h.at[pl.ds(sid * _NDPT, _NDPT)])
        pltpu.sync_copy(shden.at[pl.ds(sid * _NDPT, _NDPT)],
                        denB_h.at[pl.ds(sid * _NDPT, _NDPT)])


_pgat = functools.partial(
    pl.kernel,
    _pg_body,
    out_type=(
        jax.ShapeDtypeStruct((_NDPAD, 16), jnp.float32),
        jax.ShapeDtypeStruct((_NDPAD, 16), jnp.float32),
        jax.ShapeDtypeStruct((_NDPAD,), jnp.float32),
        jax.ShapeDtypeStruct((_NDPAD,), jnp.float32),
    ),
    mesh=_mesh,
    scratch_types=[
        pltpu.VMEM((_NB2, 128), jnp.int32),
        pltpu.VMEM((2 * _NB2, 128), jnp.int32),
        pltpu.VMEM((_NB2, 128), jnp.int32),
        pltpu.VMEM((_NB2, 128), jnp.float32),
        pltpu.VMEM((_NB2, 128), jnp.float32),
        pltpu.VMEM((_NB2 * 128,), jnp.float32),
        pltpu.VMEM((2 * _NB2 * 128,), jnp.float32),
        pltpu.VMEM((2 * _NB2 * 128, 16), jnp.float32),
        pltpu.VMEM_SHARED((_NDPAD, 16), jnp.float32),
        pltpu.VMEM_SHARED((_NDPAD,), jnp.float32),
        pltpu.SemaphoreType.DMA,
        pltpu.SemaphoreType.DMA,
    ],
    compiler_params=_sc_params,
)()


_BLK = 5000


def _t0_body(x_ref, w_ref, as_ref, ad_ref, h_ref, s_ref, t_ref):
    h = jnp.dot(x_ref[...], w_ref[...], preferred_element_type=jnp.float32)
    h_ref[...] = h
    s_ref[...] = jnp.dot(h, as_ref[...], preferred_element_type=jnp.float32)
    t_ref[...] = jnp.dot(h, ad_ref[...], preferred_element_type=jnp.float32)


def _t0(xp, Wp, a_s, a_d):
    return pl.pallas_call(
        _t0_body,
        grid=(_N // _BLK,),
        in_specs=[
            pl.BlockSpec((_BLK, 8), lambda i: (i, 0)),
            pl.BlockSpec((8, _H), lambda i: (0, 0)),
            pl.BlockSpec((_H, 1), lambda i: (0, 0)),
            pl.BlockSpec((_H, 1), lambda i: (0, 0)),
        ],
        out_specs=[
            pl.BlockSpec((_BLK, _H), lambda i: (i, 0)),
            pl.BlockSpec((_BLK, 1), lambda i: (i, 0)),
            pl.BlockSpec((_BLK, 1), lambda i: (i, 0)),
        ],
        out_shape=[
            jax.ShapeDtypeStruct((_N, _H), jnp.float32),
            jax.ShapeDtypeStruct((_N, 1), jnp.float32),
            jax.ShapeDtypeStruct((_N, 1), jnp.float32),
        ],
    )(xp, Wp, a_s, a_d)


_BLKE = _EPAD // 16


def _te_body(w4p_ref, eat_ref, out_ref):
    out_ref[...] = jnp.dot(w4p_ref[...], eat_ref[...],
                           preferred_element_type=jnp.float32)


def _te(w4p, eaT8):
    return pl.pallas_call(
        _te_body,
        grid=(_EPAD // _BLKE,),
        in_specs=[
            pl.BlockSpec((8, 8), lambda i: (0, 0)),
            pl.BlockSpec((8, _BLKE), lambda i: (0, i)),
        ],
        out_specs=pl.BlockSpec((8, _BLKE), lambda i: (0, i)),
        out_shape=jax.ShapeDtypeStruct((8, _EPAD), jnp.float32),
    )(w4p, eaT8)


def _t1_body(accA_ref, accB_ref, dena_ref, denb_ref, b_ref, w_ref, as_ref,
             ad_ref, h_ref, s_ref, t_ref, *, do_relu):
    acc = jnp.concatenate([accA_ref[...], accB_ref[...]], axis=1)
    den = jnp.maximum(dena_ref[...] + denb_ref[...], np.float32(1e-16))
    o = acc / den + b_ref[...]
    if do_relu:
        o = jnp.maximum(o, np.float32(0.0))
    h = jnp.dot(o, w_ref[...], preferred_element_type=jnp.float32)
    h_ref[...] = h
    s_ref[...] = jnp.dot(h, as_ref[...], preferred_element_type=jnp.float32)
    t_ref[...] = jnp.dot(h, ad_ref[...], preferred_element_type=jnp.float32)


def _t1(accA, accB, dena, denb, b, W, a_s, a_d, do_relu):
    return pl.pallas_call(
        functools.partial(_t1_body, do_relu=do_relu),
        grid=(_N // _BLK,),
        in_specs=[
            pl.BlockSpec((_BLK, 16), lambda i: (i, 0)),
            pl.BlockSpec((_BLK, 16), lambda i: (i, 0)),
            pl.BlockSpec((_BLK, 1), lambda i: (i, 0)),
            pl.BlockSpec((_BLK, 1), lambda i: (i, 0)),
            pl.BlockSpec((1, _H), lambda i: (0, 0)),
            pl.BlockSpec((_H, _H), lambda i: (0, 0)),
            pl.BlockSpec((_H, 1), lambda i: (0, 0)),
            pl.BlockSpec((_H, 1), lambda i: (0, 0)),
        ],
        out_specs=[
            pl.BlockSpec((_BLK, _H), lambda i: (i, 0)),
            pl.BlockSpec((_BLK, 1), lambda i: (i, 0)),
            pl.BlockSpec((_BLK, 1), lambda i: (i, 0)),
        ],
        out_shape=[
            jax.ShapeDtypeStruct((_N, _H), jnp.float32),
            jax.ShapeDtypeStruct((_N, 1), jnp.float32),
            jax.ShapeDtypeStruct((_N, 1), jnp.float32),
        ],
    )(accA, accB, dena, denb, b, W, a_s, a_d)


_BLKH = 2000
_NBH = _N // _BLKH


def _head_body(accA_ref, accB_ref, dena_ref, denb_ref, b_ref, batch_ref,
               wl_ref, bl_ref, out_ref, pooled, cnt):
    i = pl.program_id(0)

    @pl.when(i == 0)
    def _():
        pooled[...] = jnp.zeros_like(pooled)
        cnt[...] = jnp.zeros_like(cnt)

    acc = jnp.concatenate([accA_ref[...], accB_ref[...]], axis=1)
    den = jnp.maximum(dena_ref[...] + denb_ref[...], np.float32(1e-16))
    h3 = acc / den + b_ref[...]
    bb = batch_ref[0]
    gids = lax.broadcasted_iota(jnp.int32, (_G, 1), 0)
    onehotT = (gids == bb).astype(jnp.float32)
    pooled[...] += jnp.dot(onehotT, h3, preferred_element_type=jnp.float32)
    cnt[...] += jnp.dot(onehotT, jnp.ones((_BLKH, 8), jnp.float32),
                        preferred_element_type=jnp.float32)

    @pl.when(i == _NBH - 1)
    def _():
        gm = pooled[...] / jnp.maximum(cnt[..., 0:1], np.float32(1.0))
        logits = jnp.dot(gm, wl_ref[...], preferred_element_type=jnp.float32)
        logits = logits + bl_ref[...]
        mx = jnp.max(logits, axis=1, keepdims=True)
        e = jnp.exp(logits - mx)
        out_ref[...] = e / jnp.sum(e, axis=1, keepdims=True)


def _head(accA, accB, dena, denb, b, batch3, Wlp, blp):
    return pl.pallas_call(
        _head_body,
        grid=(_NBH,),
        in_specs=[
            pl.BlockSpec((_BLKH, 16), lambda i: (i, 0)),
            pl.BlockSpec((_BLKH, 16), lambda i: (i, 0)),
            pl.BlockSpec((_BLKH, 1), lambda i: (i, 0)),
            pl.BlockSpec((_BLKH, 1), lambda i: (i, 0)),
            pl.BlockSpec((1, _H), lambda i: (0, 0)),
            pl.BlockSpec((1, 1, _BLKH), lambda i: (i, 0, 0)),
            pl.BlockSpec((_H, 8), lambda i: (0, 0)),
            pl.BlockSpec((1, 8), lambda i: (0, 0)),
        ],
        out_specs=pl.BlockSpec((_G, 8), lambda i: (0, 0)),
        out_shape=jax.ShapeDtypeStruct((_G, 8), jnp.float32),
        scratch_shapes=[
            pltpu.VMEM((_G, _H), jnp.float32),
            pltpu.VMEM((_G, 8), jnp.float32),
        ],
    )(accA, accB, dena, denb, b, batch3, Wlp, blp)


def kernel(x, edge_index, edge_attr, batch, W1, a_s1, a_d1, We1, a_e1, b1,
           W2, a_s2, a_d2, We2, a_e2, b2, W3, a_s3, a_d3, We3, a_e3, b3,
           Wl, bl):
    f32 = jnp.float32
    pad_e = _EPAD - _E
    src = jnp.pad(edge_index[0], (0, pad_e)).reshape(_NROW, 128)
    dst = jnp.pad(edge_index[1], (0, pad_e)).reshape(_NROW, 128)
    eaT8 = jnp.pad(jnp.pad(edge_attr, ((0, pad_e), (0, 0))).T,
                   ((0, 4), (0, 0)))
    xp = jnp.pad(x, ((0, 0), (0, 1)))
    W1p = jnp.pad(W1, ((0, 1), (0, 0)))
    batch3 = batch.reshape(_NBH, 1, _BLKH)
    Wlp = jnp.pad(Wl, ((0, 0), (0, 6)))
    blp = jnp.concatenate([bl, jnp.full((6,), -1e30, f32)]).reshape(1, 8)

    w4p = jnp.zeros((8, 8), f32)
    w4p = w4p.at[0, :4].set(We1 @ a_e1)
    w4p = w4p.at[1, :4].set(We2 @ a_e2)
    w4p = w4p.at[2, :4].set(We3 @ a_e3)
    et8 = _te(w4p, eaT8)

    layers = [
        (W1p, a_s1, a_d1, b1),
        (W2, a_s2, a_d2, b2),
        (W3, a_s3, a_d3, b3),
    ]

    h, s, t = _t0(xp, W1p, a_s1.reshape(_H, 1), a_d1.reshape(_H, 1))
    for li in range(3):
        _, _, _, b = layers[li]
        accA, accB, denA, denB = _pgat(h.reshape(2 * _N, 16), src, dst,
                                       et8[li], s.reshape(_N), t.reshape(_N))
        accA, accB = accA[:_N], accB[:_N]
        denA2 = denA[:_N].reshape(_N, 1)
        denB2 = denB[:_N].reshape(_N, 1)
        if li < 2:
            Wn, a_sn, a_dn, _ = layers[li + 1]
            h, s, t = _t1(accA, accB, denA2, denB2, b.reshape(1, _H), Wn,
                          a_sn.reshape(_H, 1), a_dn.reshape(_H, 1),
                          do_relu=True)
        else:
            out = _head(accA, accB, denA2, denB2, b.reshape(1, _H), batch3,
                        Wlp, blp)
    return out[:, :2]


# speedup vs baseline: 1.0433x; 1.0420x over previous
import functools

import jax
import jax.numpy as jnp
import numpy as np
from jax import lax
from jax.experimental import pallas as pl
from jax.experimental.pallas import tpu as pltpu
from jax.experimental.pallas import tpu_sc as plsc

_N = 100000
_E = 1600000
_G = 512
_H = 32

_EPAD = 1605632
_NROW = _EPAD // 128

_PT1_ROWS = _NROW // 32
_NB1 = 8
_NC1 = _PT1_ROWS // _NB1

_PT2_ROWS = _NROW // 16
_NB2 = 4
_NC2 = _PT2_ROWS // _NB2

_NDPAD = 100096
_NDPT = _NDPAD // 16

_mesh = plsc.VectorSubcoreMesh(core_axis_name="c", subcore_axis_name="s")
_sc_params = pltpu.CompilerParams(use_tc_tiling_on_sc=False)


def _pg_body(hrows_h, src_h, dst_h, et_h, s_h, t_h,
             accA_h, accB_h, denA_h,
             srcv, dstv, gidxv, sv, tv, etv, exv, rowsbuf,
             shacc, shden, semg, sems):
    c = lax.axis_index("c")
    sid = lax.axis_index("s")
    lanes = lax.iota(jnp.int32, 16)

    def zfill(i, _):
        rowsbuf[i] = jnp.zeros((16,), jnp.float32)
        return 0
    lax.fori_loop(0, 368, zfill, 0)

    def zfilld(i, _):
        exv[pl.ds(i * 16, 16)] = jnp.zeros((16,), jnp.float32)
        return 0
    lax.fori_loop(0, 23, zfilld, 0)

    zb = rowsbuf.at[pl.ds(0, 368)]
    zbd = exv.at[pl.ds(0, 368)]

    def zcopy(j, _):
        pltpu.sync_copy(zb, shacc.at[pl.ds(sid * _NDPT + j * 368, 368)])
        pltpu.sync_copy(zbd, shden.at[pl.ds(sid * _NDPT + j * 368, 368)])
        return 0
    lax.fori_loop(0, 17, zcopy, 0)
    plsc.subcore_barrier()

    def stage_fire(nc, hh):
        r0 = sid * _PT2_ROWS + nc * _NB2
        pltpu.sync_copy(src_h.at[pl.ds(r0, _NB2)], srcv)
        pltpu.sync_copy(dst_h.at[pl.ds(r0, _NB2)],
                        dstv.at[pl.ds(hh * _NB2, _NB2)])
        pltpu.async_copy(et_h.at[pl.ds(r0 * 128, _NB2 * 128)], etv, semg)
        for b in range(_NB2):
            pltpu.async_copy(s_h.at[srcv.at[b]], sv.at[b], semg)
            pltpu.async_copy(t_h.at[dstv.at[hh * _NB2 + b]], tv.at[b], semg)
        for b in range(_NB2):
            for g in range(8):
                sl = pl.ds(g * 16, 16)
                gidxv[b, sl] = srcv[b, sl] * 2 + c
        for b in range(_NB2):
            pltpu.async_copy(
                hrows_h.at[gidxv.at[b]],
                rowsbuf.at[pl.ds(hh * (_NB2 * 128) + b * 128, 128)], semg)

    def wait_gathers():
        pltpu.make_async_copy(
            et_h.at[pl.ds(0, _NB2 * 128)], etv, semg).wait()
        for b in range(_NB2):
            pltpu.make_async_copy(s_h.at[srcv.at[b]], sv.at[b], semg).wait()
            pltpu.make_async_copy(s_h.at[srcv.at[b]], tv.at[b], semg).wait()
        for b in range(_NB2):
            pltpu.make_async_copy(
                hrows_h.at[gidxv.at[b]],
                rowsbuf.at[pl.ds(b * 128, 128)], semg).wait()

    def drain_scatters(hh):
        for b in range(_NB2):
            pltpu.make_async_copy(
                rowsbuf.at[pl.ds(hh * (_NB2 * 128) + b * 128, 128)],
                shacc.at[dstv.at[hh * _NB2 + b]], sems).wait()

        @pl.when(c == 0)
        def _():
            for b in range(_NB2):
                pltpu.make_async_copy(
                    exv.at[pl.ds(hh * (_NB2 * 128) + b * 128, 128)],
                    shden.at[dstv.at[hh * _NB2 + b]], sems).wait()

    stage_fire(0, 0)

    def chunk(ci, _):
        hh = lax.rem(ci, 2)
        ho = hh * (_NB2 * 128)
        r0 = sid * _PT2_ROWS + ci * _NB2
        wait_gathers()
        for b in range(_NB2):
            for g in range(8):
                sl16 = pl.ds(g * 16, 16)
                a = sv[b, sl16] + tv[b, sl16] + etv[pl.ds(b * 128 + g * 16, 16)]
                a = jnp.where(a > 0, a, a * np.float32(0.2))
                ge = (r0 + b) * 128 + g * 16 + lanes
                exv[pl.ds(ho + b * 128 + g * 16, 16)] = jnp.where(
                    ge < _E, jnp.exp(a), np.float32(0.0))

        def scale(q, _):
            exq = exv[pl.ds(ho + q * 16, 16)]
            for l in range(16):
                row = ho + q * 16 + l
                rowsbuf[row] = rowsbuf[row] * exq[l]
            return 0
        lax.fori_loop(0, _NB2 * 8, scale, 0)

        @pl.when(ci > 0)
        def _():
            drain_scatters(1 - hh)

        for b in range(_NB2):
            pltpu.async_copy(rowsbuf.at[pl.ds(ho + b * 128, 128)],
                             shacc.at[dstv.at[hh * _NB2 + b]], sems, add=True)

        @pl.when(c == 0)
        def _():
            for b in range(_NB2):
                pltpu.async_copy(exv.at[pl.ds(ho + b * 128, 128)],
                                 shden.at[dstv.at[hh * _NB2 + b]], sems,
                                 add=True)

        @pl.when(ci + 1 < _NC2)
        def _():
            stage_fire(ci + 1, 1 - hh)
        return 0

    lax.fori_loop(0, _NC2, chunk, 0)
    drain_scatters(lax.rem(_NC2 - 1, 2))
    plsc.subcore_barrier()

    @pl.when(c == 0)
    def _():
        pltpu.sync_copy(shacc.at[pl.ds(sid * _NDPT, _NDPT)],
                        accA_h.at[pl.ds(sid * _NDPT, _NDPT)])
        pltpu.sync_copy(shden.at[pl.ds(sid * _NDPT, _NDPT)],
                        denA_h.at[pl.ds(sid * _NDPT, _NDPT)])

    @pl.when(c == 1)
    def _():
        pltpu.sync_copy(shacc.at[pl.ds(sid * _NDPT, _NDPT)],
                        accB_h.at[pl.ds(sid * _NDPT, _NDPT)])


_pgat = functools.partial(
    pl.kernel,
    _pg_body,
    out_type=(
        jax.ShapeDtypeStruct((_NDPAD, 16), jnp.float32),
        jax.ShapeDtypeStruct((_NDPAD, 16), jnp.float32),
        jax.ShapeDtypeStruct((_NDPAD,), jnp.float32),
    ),
    mesh=_mesh,
    scratch_types=[
        pltpu.VMEM((_NB2, 128), jnp.int32),
        pltpu.VMEM((2 * _NB2, 128), jnp.int32),
        pltpu.VMEM((_NB2, 128), jnp.int32),
        pltpu.VMEM((_NB2, 128), jnp.float32),
        pltpu.VMEM((_NB2, 128), jnp.float32),
        pltpu.VMEM((_NB2 * 128,), jnp.float32),
        pltpu.VMEM((2 * _NB2 * 128,), jnp.float32),
        pltpu.VMEM((2 * _NB2 * 128, 16), jnp.float32),
        pltpu.VMEM_SHARED((_NDPAD, 16), jnp.float32),
        pltpu.VMEM_SHARED((_NDPAD,), jnp.float32),
        pltpu.SemaphoreType.DMA,
        pltpu.SemaphoreType.DMA,
    ],
    compiler_params=_sc_params,
)()


_BLK = 5000


def _t0_body(x_ref, w_ref, as_ref, ad_ref, h_ref, s_ref, t_ref):
    h = jnp.dot(x_ref[...], w_ref[...], preferred_element_type=jnp.float32)
    h_ref[...] = h
    s_ref[...] = jnp.dot(h, as_ref[...], preferred_element_type=jnp.float32)
    t_ref[...] = jnp.dot(h, ad_ref[...], preferred_element_type=jnp.float32)


def _t0(xp, Wp, a_s, a_d):
    return pl.pallas_call(
        _t0_body,
        grid=(_N // _BLK,),
        in_specs=[
            pl.BlockSpec((_BLK, 8), lambda i: (i, 0)),
            pl.BlockSpec((8, _H), lambda i: (0, 0)),
            pl.BlockSpec((_H, 1), lambda i: (0, 0)),
            pl.BlockSpec((_H, 1), lambda i: (0, 0)),
        ],
        out_specs=[
            pl.BlockSpec((_BLK, _H), lambda i: (i, 0)),
            pl.BlockSpec((_BLK, 1), lambda i: (i, 0)),
            pl.BlockSpec((_BLK, 1), lambda i: (i, 0)),
        ],
        out_shape=[
            jax.ShapeDtypeStruct((_N, _H), jnp.float32),
            jax.ShapeDtypeStruct((_N, 1), jnp.float32),
            jax.ShapeDtypeStruct((_N, 1), jnp.float32),
        ],
    )(xp, Wp, a_s, a_d)


_BLKE = _EPAD // 16


def _te_body(w4p_ref, eat_ref, out_ref):
    out_ref[...] = jnp.dot(w4p_ref[...], eat_ref[...],
                           preferred_element_type=jnp.float32)


def _te(w4p, eaT8):
    return pl.pallas_call(
        _te_body,
        grid=(_EPAD // _BLKE,),
        in_specs=[
            pl.BlockSpec((8, 8), lambda i: (0, 0)),
            pl.BlockSpec((8, _BLKE), lambda i: (0, i)),
        ],
        out_specs=pl.BlockSpec((8, _BLKE), lambda i: (0, i)),
        out_shape=jax.ShapeDtypeStruct((8, _EPAD), jnp.float32),
    )(w4p, eaT8)


def _t1_body(accA_ref, accB_ref, dena_ref, b_ref, w_ref, as_ref,
             ad_ref, h_ref, s_ref, t_ref, *, do_relu):
    acc = jnp.concatenate([accA_ref[...], accB_ref[...]], axis=1)
    den = jnp.maximum(dena_ref[...], np.float32(1e-16))
    o = acc / den + b_ref[...]
    if do_relu:
        o = jnp.maximum(o, np.float32(0.0))
    h = jnp.dot(o, w_ref[...], preferred_element_type=jnp.float32)
    h_ref[...] = h
    s_ref[...] = jnp.dot(h, as_ref[...], preferred_element_type=jnp.float32)
    t_ref[...] = jnp.dot(h, ad_ref[...], preferred_element_type=jnp.float32)


def _t1(accA, accB, dena, b, W, a_s, a_d, do_relu):
    return pl.pallas_call(
        functools.partial(_t1_body, do_relu=do_relu),
        grid=(_N // _BLK,),
        in_specs=[
            pl.BlockSpec((_BLK, 16), lambda i: (i, 0)),
            pl.BlockSpec((_BLK, 16), lambda i: (i, 0)),
            pl.BlockSpec((_BLK, 1), lambda i: (i, 0)),
            pl.BlockSpec((1, _H), lambda i: (0, 0)),
            pl.BlockSpec((_H, _H), lambda i: (0, 0)),
            pl.BlockSpec((_H, 1), lambda i: (0, 0)),
            pl.BlockSpec((_H, 1), lambda i: (0, 0)),
        ],
        out_specs=[
            pl.BlockSpec((_BLK, _H), lambda i: (i, 0)),
            pl.BlockSpec((_BLK, 1), lambda i: (i, 0)),
            pl.BlockSpec((_BLK, 1), lambda i: (i, 0)),
        ],
        out_shape=[
            jax.ShapeDtypeStruct((_N, _H), jnp.float32),
            jax.ShapeDtypeStruct((_N, 1), jnp.float32),
            jax.ShapeDtypeStruct((_N, 1), jnp.float32),
        ],
    )(accA, accB, dena, b, W, a_s, a_d)


_BLKH = 2000
_NBH = _N // _BLKH


def _head_body(accA_ref, accB_ref, dena_ref, b_ref, batch_ref,
               wl_ref, bl_ref, out_ref, pooled, cnt):
    i = pl.program_id(0)

    @pl.when(i == 0)
    def _():
        pooled[...] = jnp.zeros_like(pooled)
        cnt[...] = jnp.zeros_like(cnt)

    acc = jnp.concatenate([accA_ref[...], accB_ref[...]], axis=1)
    den = jnp.maximum(dena_ref[...], np.float32(1e-16))
    h3 = acc / den + b_ref[...]
    bb = batch_ref[0]
    gids = lax.broadcasted_iota(jnp.int32, (_G, 1), 0)
    onehotT = (gids == bb).astype(jnp.float32)
    pooled[...] += jnp.dot(onehotT, h3, preferred_element_type=jnp.float32)
    cnt[...] += jnp.dot(onehotT, jnp.ones((_BLKH, 8), jnp.float32),
                        preferred_element_type=jnp.float32)

    @pl.when(i == _NBH - 1)
    def _():
        gm = pooled[...] / jnp.maximum(cnt[..., 0:1], np.float32(1.0))
        logits = jnp.dot(gm, wl_ref[...], preferred_element_type=jnp.float32)
        logits = logits + bl_ref[...]
        mx = jnp.max(logits, axis=1, keepdims=True)
        e = jnp.exp(logits - mx)
        out_ref[...] = e / jnp.sum(e, axis=1, keepdims=True)


def _head(accA, accB, dena, b, batch3, Wlp, blp):
    return pl.pallas_call(
        _head_body,
        grid=(_NBH,),
        in_specs=[
            pl.BlockSpec((_BLKH, 16), lambda i: (i, 0)),
            pl.BlockSpec((_BLKH, 16), lambda i: (i, 0)),
            pl.BlockSpec((_BLKH, 1), lambda i: (i, 0)),
            pl.BlockSpec((1, _H), lambda i: (0, 0)),
            pl.BlockSpec((1, 1, _BLKH), lambda i: (i, 0, 0)),
            pl.BlockSpec((_H, 8), lambda i: (0, 0)),
            pl.BlockSpec((1, 8), lambda i: (0, 0)),
        ],
        out_specs=pl.BlockSpec((_G, 8), lambda i: (0, 0)),
        out_shape=jax.ShapeDtypeStruct((_G, 8), jnp.float32),
        scratch_shapes=[
            pltpu.VMEM((_G, _H), jnp.float32),
            pltpu.VMEM((_G, 8), jnp.float32),
        ],
    )(accA, accB, dena, b, batch3, Wlp, blp)


def kernel(x, edge_index, edge_attr, batch, W1, a_s1, a_d1, We1, a_e1, b1,
           W2, a_s2, a_d2, We2, a_e2, b2, W3, a_s3, a_d3, We3, a_e3, b3,
           Wl, bl):
    f32 = jnp.float32
    pad_e = _EPAD - _E
    src = jnp.pad(edge_index[0], (0, pad_e)).reshape(_NROW, 128)
    dst = jnp.pad(edge_index[1], (0, pad_e)).reshape(_NROW, 128)
    eaT8 = jnp.pad(jnp.pad(edge_attr, ((0, pad_e), (0, 0))).T,
                   ((0, 4), (0, 0)))
    xp = jnp.pad(x, ((0, 0), (0, 1)))
    W1p = jnp.pad(W1, ((0, 1), (0, 0)))
    batch3 = batch.reshape(_NBH, 1, _BLKH)
    Wlp = jnp.pad(Wl, ((0, 0), (0, 6)))
    blp = jnp.concatenate([bl, jnp.full((6,), -1e30, f32)]).reshape(1, 8)

    w4p = jnp.zeros((8, 8), f32)
    w4p = w4p.at[0, :4].set(We1 @ a_e1)
    w4p = w4p.at[1, :4].set(We2 @ a_e2)
    w4p = w4p.at[2, :4].set(We3 @ a_e3)
    et8 = _te(w4p, eaT8)

    layers = [
        (W1p, a_s1, a_d1, b1),
        (W2, a_s2, a_d2, b2),
        (W3, a_s3, a_d3, b3),
    ]

    h, s, t = _t0(xp, W1p, a_s1.reshape(_H, 1), a_d1.reshape(_H, 1))
    for li in range(3):
        _, _, _, b = layers[li]
        accA, accB, denA = _pgat(h.reshape(2 * _N, 16), src, dst,
                                 et8[li], s.reshape(_N), t.reshape(_N))
        accA, accB = accA[:_N], accB[:_N]
        denA2 = denA[:_N].reshape(_N, 1)
        if li < 2:
            Wn, a_sn, a_dn, _ = layers[li + 1]
            h, s, t = _t1(accA, accB, denA2, b.reshape(1, _H), Wn,
                          a_sn.reshape(_H, 1), a_dn.reshape(_H, 1),
                          do_relu=True)
        else:
            out = _head(accA, accB, denA2, b.reshape(1, _H), batch3,
                        Wlp, blp)
    return out[:, :2]
